# baseline (speedup 1.0000x reference)
import functools

import jax
import jax.numpy as jnp
from jax import lax
from jax.experimental import pallas as pl
from jax.experimental.pallas import tpu as pltpu
from jax.experimental.pallas import tpu_sc as plsc

B = 16384
D = 64
V = 1000000
NC = 2
NS = 16
L = 16
NW = NC * NS
BPW = B // NW
XTAIL = (V // 128) * 128
NBLK = V // 128 + 1
CAP = 768
KR = 12


def _extract_body(u_h, v_h, eu_h, ev_h, tu_h, tv_h, su_h, sv_h,
                  idxbuf, tailbuf, blk, rowstage, cntv, offs, cur, ex, tot,
                  sem_b, sem_w):
    wid = lax.axis_index("s") * NC + lax.axis_index("c")
    bstart = (NBLK * wid) // NW
    bend = (NBLK * (wid + 1)) // NW
    lo = bstart * 128
    hi = bend * 128
    lanes = lax.iota(jnp.int32, L)

    def wait_block():
        pltpu.make_async_copy(eu_h.at[pl.ds(0, D), pl.ds(0, 128)],
                              blk.at[0], sem_b).wait()

    def wait_write():
        pltpu.make_async_copy(su_h.at[0], rowstage.at[0], sem_w).wait()

    def do_table(tab_h, tail_h, stag_h, idx_h):
        pltpu.sync_copy(idx_h, idxbuf)
        pltpu.sync_copy(tail_h, tailbuf)

        def zero(i, _):
            cntv[pl.ds(16 * i, L)] = jnp.zeros((L,), jnp.int32)
            return 0

        lax.fori_loop(0, 16, zero, 0)
        tot[0] = 0
        ones = jnp.ones((L,), jnp.int32)

        def count_chunk(c, _):
            vx = idxbuf[pl.ds(16 * c, L)]
            m = (vx >= lo) & (vx < hi)
            npop = plsc.all_reduce_population_count(m)[0]

            @pl.when(npop > 0)
            def _():
                j = jnp.clip((vx >> 7) - bstart, 0, 255)
                plsc.addupdate_scatter(cntv, [j], ones, mask=m)

            tot[0] = tot[0] + npop
            return 0

        lax.fori_loop(0, B // L, count_chunk, 0)

        def spill(i, _):
            vc = cntv[pl.ds(16 * i, L)]
            for k in range(L):
                cur[16 * i + k] = vc[k]
            return 0

        lax.fori_loop(0, 16, spill, 0)

        offs[0] = 0

        def scan(i, _):
            offs[i + 1] = offs[i] + cur[i]
            return 0

        lax.fori_loop(0, 256, scan, 0)

        def reset(i, _):
            cur[i] = offs[i]
            return 0

        lax.fori_loop(0, 256, reset, 0)

        total = tot[0]

        @pl.when(total <= CAP)
        def _fast():
            def fill_chunk(c, _):
                vx = idxbuf[pl.ds(16 * c, L)]
                m = (vx >= lo) & (vx < hi)
                npop = plsc.all_reduce_population_count(m)[0]

                @pl.when(npop > 0)
                def _():
                    for k in range(L):
                        x = vx[k]

                        @pl.when((x >= lo) & (x < hi))
                        def _():
                            j = (x >> 7) - bstart
                            s = cur[j]
                            cur[j] = s + 1
                            ex[s] = (16 * c + k) | \
                                (jnp.bitwise_and(x, 127) << 14)
                return 0

            lax.fori_loop(0, B // L, fill_chunk, 0)

            nbm = jnp.minimum(bend, NBLK - 1) - bstart

            def fire(p):
                @pl.when(offs[p + 1] > offs[p])
                def _():
                    jj = bstart + p
                    off = pl.multiple_of(jj * 128, 128)
                    pltpu.async_copy(
                        tab_h.at[pl.ds(0, D), pl.ds(off, 128)],
                        blk.at[jnp.mod(p, KR)], sem_b)

            for p in range(KR):
                @pl.when(p < nbm)
                def _(p=p):
                    fire(p)

            def extract_to_stage(e, s, src, src_r):
                pos = jnp.bitwise_and(e, 16383)
                col = jnp.full((L,), jnp.bitwise_and(e >> 14, 127),
                               jnp.int32)
                slot = jnp.bitwise_and(s, 15)
                for q in range(D // L):
                    if src_r is None:
                        vq = plsc.load_gather(src, [q * L + lanes, col])
                    else:
                        vq = plsc.load_gather(
                            src, [src_r, q * L + lanes, col])
                    rowstage[slot, pl.ds(q * L, L)] = vq
                return pos, slot

            def blk_loop(bi, _):
                @pl.when(offs[bi + 1] > offs[bi])
                def _():
                    wait_block()
                    rbuf = jnp.full((L,), jnp.mod(bi, KR),
                                    jnp.int32)

                    def ent(s, _):
                        @pl.when(s >= 16)
                        def _():
                            wait_write()
                        pos, slot = extract_to_stage(ex[s], s, blk, rbuf)
                        pltpu.async_copy(rowstage.at[slot],
                                         stag_h.at[pos], sem_w)
                        return 0

                    lax.fori_loop(offs[bi], offs[bi + 1], ent, 0)

                @pl.when(bi + KR < nbm)
                def _():
                    fire(bi + KR)
                return 0

            lax.fori_loop(0, nbm, blk_loop, 0)

            @pl.when(bend == NBLK)
            def _():
                jt = NBLK - 1 - bstart

                def tent(s, _):
                    @pl.when(s >= 16)
                    def _():
                        wait_write()
                    pos, slot = extract_to_stage(ex[s], s, tailbuf, None)
                    pltpu.async_copy(rowstage.at[slot], stag_h.at[pos],
                                     sem_w)
                    return 0

                lax.fori_loop(offs[jt], offs[jt + 1], tent, 0)

            def drain(i, _):
                wait_write()
                return 0

            lax.fori_loop(0, jnp.minimum(total, 16), drain, 0)

        @pl.when(total > CAP)
        def _slow():
            def slow_chunk(c, _):
                vx = idxbuf[pl.ds(16 * c, L)]
                for k in range(L):
                    x = vx[k]

                    @pl.when((x >= lo) & (x < hi) & (x < XTAIL))
                    def _():
                        jj = x >> 7
                        off = pl.multiple_of(jj * 128, 128)
                        pltpu.async_copy(
                            tab_h.at[pl.ds(0, D), pl.ds(off, 128)],
                            blk.at[0], sem_b)
                        wait_block()
                        e = (16 * c + k) | (jnp.bitwise_and(x, 127) << 14)
                        pos, slot = extract_to_stage2(e, blk)
                        pltpu.async_copy(rowstage.at[slot],
                                         stag_h.at[pos], sem_w)
                        wait_write()

                    @pl.when((x >= lo) & (x < hi) & (x >= XTAIL))
                    def _():
                        e = (16 * c + k) | (jnp.bitwise_and(x, 127) << 14)
                        pos, slot = extract_to_stage2(e, None)
                        pltpu.async_copy(rowstage.at[slot],
                                         stag_h.at[pos], sem_w)
                        wait_write()
                return 0

            def extract_to_stage2(e, src):
                pos = jnp.bitwise_and(e, 16383)
                col = jnp.full((L,), jnp.bitwise_and(e >> 14, 127),
                               jnp.int32)
                zero16 = jnp.zeros((L,), jnp.int32)
                for q in range(D // L):
                    if src is None:
                        vq = plsc.load_gather(tailbuf,
                                              [q * L + lanes, col])
                    else:
                        vq = plsc.load_gather(
                            src, [zero16, q * L + lanes, col])
                    rowstage[0, pl.ds(q * L, L)] = vq
                return pos, 0

            lax.fori_loop(0, B // L, slow_chunk, 0)

    do_table(eu_h, tu_h, su_h, u_h)
    do_table(ev_h, tv_h, sv_h, v_h)


def _combine_body(u2_h, v2_h, su_h, sv_h, bu_h, bv_h, out_h,
                  idx_u, idx_v, sbu, sbv, b_u, b_v, out_v, sem):
    wid = lax.axis_index("s") * NC + lax.axis_index("c")
    base = wid * BPW
    lanes = lax.iota(jnp.int32, L)

    pltpu.sync_copy(u2_h.at[pl.ds(wid * 4, 4)], idx_u)
    pltpu.sync_copy(v2_h.at[pl.ds(wid * 4, 4)], idx_v)

    copies = []
    for j in range(4):
        dst = pl.ds(j * 128, 128)
        copies.append(pltpu.async_copy(bu_h.at[idx_u.at[j]],
                                       b_u.at[dst], sem))
        copies.append(pltpu.async_copy(bv_h.at[idx_v.at[j]],
                                       b_v.at[dst], sem))
    for c in copies:
        c.wait()

    for h in range(2):
        hb = base + h * 256
        pltpu.sync_copy(su_h.at[pl.ds(hb, 256)], sbu)
        pltpu.sync_copy(sv_h.at[pl.ds(hb, 256)], sbv)

        def group(g, _):
            rows = g * L + lanes
            boff = h * 256 + g * L
            acc = b_u[pl.ds(boff, L)] + b_v[pl.ds(boff, L)]
            for d in range(D):
                col = jnp.full((L,), d, jnp.int32)
                acc = acc + plsc.load_gather(sbu, [rows, col]) * \
                    plsc.load_gather(sbv, [rows, col])
            out_v[pl.ds(h * 256 + g * L, L)] = jnp.clip(acc, 0.0, 5.0)
            return 0

        lax.fori_loop(0, 256 // L, group, 0)

    pltpu.sync_copy(out_v, out_h.at[pl.ds(base, BPW)])


_extract = functools.partial(
    pl.kernel,
    out_type=(jax.ShapeDtypeStruct((B, 128), jnp.float32),
              jax.ShapeDtypeStruct((B, 128), jnp.float32)),
    mesh=plsc.VectorSubcoreMesh(core_axis_name="c", subcore_axis_name="s"),
    scratch_types=[
        pltpu.VMEM((B,), jnp.int32),
        pltpu.VMEM((D, 64), jnp.float32),
        pltpu.VMEM((KR, D, 128), jnp.float32),
        pltpu.VMEM((16, 128), jnp.float32),
        pltpu.VMEM((256,), jnp.int32),
        pltpu.SMEM((257,), jnp.int32),
        pltpu.SMEM((256,), jnp.int32),
        pltpu.SMEM((CAP,), jnp.int32),
        pltpu.SMEM((8,), jnp.int32),
        pltpu.SemaphoreType.DMA,
        pltpu.SemaphoreType.DMA,
    ],
    compiler_params=pltpu.CompilerParams(needs_layout_passes=False),
)(_extract_body)


_combine = functools.partial(
    pl.kernel,
    out_type=jax.ShapeDtypeStruct((B,), jnp.float32),
    mesh=plsc.VectorSubcoreMesh(core_axis_name="c", subcore_axis_name="s"),
    scratch_types=[
        pltpu.VMEM((4, 128), jnp.int32),
        pltpu.VMEM((4, 128), jnp.int32),
        pltpu.VMEM((256, 128), jnp.float32),
        pltpu.VMEM((256, 128), jnp.float32),
        pltpu.VMEM((BPW,), jnp.float32),
        pltpu.VMEM((BPW,), jnp.float32),
        pltpu.VMEM((BPW,), jnp.float32),
        pltpu.SemaphoreType.DMA,
    ],
    compiler_params=pltpu.CompilerParams(needs_layout_passes=False),
)(_combine_body)


def kernel(user, item, embed_user, embed_item, bias_user, bias_item):
    u1 = user.astype(jnp.int32)
    v1 = item.astype(jnp.int32)
    eu_t = embed_user.T
    ev_t = embed_item.T
    tail_u = embed_user[XTAIL:].T
    tail_v = embed_item[XTAIL:].T
    su, sv = _extract(u1, v1, eu_t, ev_t, tail_u, tail_v)
    return _combine(u1.reshape(128, 128), v1.reshape(128, 128), su, sv,
                    bias_user.reshape(-1), bias_item.reshape(-1))


# speedup vs baseline: 1.0213x; 1.0213x over previous
import functools

import jax
import jax.numpy as jnp
from jax import lax
from jax.experimental import pallas as pl
from jax.experimental.pallas import tpu as pltpu
from jax.experimental.pallas import tpu_sc as plsc

B = 16384
D = 64
V = 1000000
NC = 2
NS = 16
L = 16
NW = NC * NS
BPW = B // NW
XTAIL = (V // 128) * 128
NBLK = V // 128 + 1
CAP = 768
KR = 8


def _extract_body(u_h, v_h, eu_h, ev_h, tu_h, tv_h, su_h, sv_h,
                  idxbuf, tailbuf, blk, rowstage, cntv, offs, cur, ex, tot,
                  sem_b, sem_w):
    wid = lax.axis_index("s") * NC + lax.axis_index("c")
    bstart = (NBLK * wid) // NW
    bend = (NBLK * (wid + 1)) // NW
    lo = bstart * 128
    hi = bend * 128
    lanes = lax.iota(jnp.int32, L)

    def wait_block():
        pltpu.make_async_copy(eu_h.at[pl.ds(0, D), pl.ds(0, 128)],
                              blk.at[0], sem_b).wait()

    def wait_write():
        pltpu.make_async_copy(su_h.at[0], rowstage.at[0, pl.ds(0, D)],
                              sem_w).wait()

    def do_table(tab_h, tail_h, stag_h, idx_h):
        pltpu.sync_copy(idx_h, idxbuf)
        pltpu.sync_copy(tail_h, tailbuf)

        def zero(i, _):
            cntv[pl.ds(16 * i, L)] = jnp.zeros((L,), jnp.int32)
            return 0

        lax.fori_loop(0, 16, zero, 0)
        tot[0] = 0
        ones = jnp.ones((L,), jnp.int32)

        def count_chunk(c, _):
            vx = idxbuf[pl.ds(16 * c, L)]
            m = (vx >= lo) & (vx < hi)
            npop = plsc.all_reduce_population_count(m)[0]

            @pl.when(npop > 0)
            def _():
                j = jnp.clip((vx >> 7) - bstart, 0, 255)
                plsc.addupdate_scatter(cntv, [j], ones, mask=m)

            tot[0] = tot[0] + npop
            return 0

        lax.fori_loop(0, B // L, count_chunk, 0)

        def spill(i, _):
            vc = cntv[pl.ds(16 * i, L)]
            for k in range(L):
                cur[16 * i + k] = vc[k]
            return 0

        lax.fori_loop(0, 16, spill, 0)

        offs[0] = 0

        def scan(i, _):
            offs[i + 1] = offs[i] + cur[i]
            return 0

        lax.fori_loop(0, 256, scan, 0)

        def reset(i, _):
            cur[i] = offs[i]
            return 0

        lax.fori_loop(0, 256, reset, 0)

        total = tot[0]

        @pl.when(total <= CAP)
        def _fast():
            def fill_chunk(c, _):
                vx = idxbuf[pl.ds(16 * c, L)]
                m = (vx >= lo) & (vx < hi)
                npop = plsc.all_reduce_population_count(m)[0]

                @pl.when(npop > 0)
                def _():
                    for k in range(L):
                        x = vx[k]

                        @pl.when((x >= lo) & (x < hi))
                        def _():
                            j = (x >> 7) - bstart
                            s = cur[j]
                            cur[j] = s + 1
                            ex[s] = (16 * c + k) | \
                                (jnp.bitwise_and(x, 127) << 14)
                return 0

            lax.fori_loop(0, B // L, fill_chunk, 0)

            nbm = jnp.minimum(bend, NBLK - 1) - bstart

            def fire(p):
                @pl.when(offs[p + 1] > offs[p])
                def _():
                    jj = bstart + p
                    off = pl.multiple_of(jj * 128, 128)
                    pltpu.async_copy(
                        tab_h.at[pl.ds(0, D), pl.ds(off, 128)],
                        blk.at[jnp.bitwise_and(p, KR - 1)], sem_b)

            for p in range(KR):
                @pl.when(p < nbm)
                def _(p=p):
                    fire(p)

            def extract_to_stage(e, s, src, src_r):
                pos = jnp.bitwise_and(e, 16383)
                col = jnp.full((L,), jnp.bitwise_and(e >> 14, 127),
                               jnp.int32)
                slot = jnp.bitwise_and(s, 15)
                for q in range(D // L):
                    if src_r is None:
                        vq = plsc.load_gather(src, [q * L + lanes, col])
                    else:
                        vq = plsc.load_gather(
                            src, [src_r, q * L + lanes, col])
                    rowstage[slot, pl.ds(q * L, L)] = vq
                return pos, slot

            def blk_loop(bi, _):
                @pl.when(offs[bi + 1] > offs[bi])
                def _():
                    wait_block()
                    rbuf = jnp.full((L,), jnp.bitwise_and(bi, KR - 1),
                                    jnp.int32)

                    def ent(s, _):
                        @pl.when(s >= 16)
                        def _():
                            wait_write()
                        pos, slot = extract_to_stage(ex[s], s, blk, rbuf)
                        pltpu.async_copy(rowstage.at[slot, pl.ds(0, D)],
                                         stag_h.at[pos], sem_w)
                        return 0

                    lax.fori_loop(offs[bi], offs[bi + 1], ent, 0)

                @pl.when(bi + KR < nbm)
                def _():
                    fire(bi + KR)
                return 0

            lax.fori_loop(0, nbm, blk_loop, 0)

            @pl.when(bend == NBLK)
            def _():
                jt = NBLK - 1 - bstart

                def tent(s, _):
                    @pl.when(s >= 16)
                    def _():
                        wait_write()
                    pos, slot = extract_to_stage(ex[s], s, tailbuf, None)
                    pltpu.async_copy(rowstage.at[slot, pl.ds(0, D)],
                                     stag_h.at[pos], sem_w)
                    return 0

                lax.fori_loop(offs[jt], offs[jt + 1], tent, 0)

            def drain(i, _):
                wait_write()
                return 0

            lax.fori_loop(0, jnp.minimum(total, 16), drain, 0)

        @pl.when(total > CAP)
        def _slow():
            def slow_chunk(c, _):
                vx = idxbuf[pl.ds(16 * c, L)]
                for k in range(L):
                    x = vx[k]

                    @pl.when((x >= lo) & (x < hi) & (x < XTAIL))
                    def _():
                        jj = x >> 7
                        off = pl.multiple_of(jj * 128, 128)
                        pltpu.async_copy(
                            tab_h.at[pl.ds(0, D), pl.ds(off, 128)],
                            blk.at[0], sem_b)
                        wait_block()
                        e = (16 * c + k) | (jnp.bitwise_and(x, 127) << 14)
                        pos, slot = extract_to_stage2(e, blk)
                        pltpu.async_copy(rowstage.at[slot, pl.ds(0, D)],
                                         stag_h.at[pos], sem_w)
                        wait_write()

                    @pl.when((x >= lo) & (x < hi) & (x >= XTAIL))
                    def _():
                        e = (16 * c + k) | (jnp.bitwise_and(x, 127) << 14)
                        pos, slot = extract_to_stage2(e, None)
                        pltpu.async_copy(rowstage.at[slot, pl.ds(0, D)],
                                         stag_h.at[pos], sem_w)
                        wait_write()
                return 0

            def extract_to_stage2(e, src):
                pos = jnp.bitwise_and(e, 16383)
                col = jnp.full((L,), jnp.bitwise_and(e >> 14, 127),
                               jnp.int32)
                zero16 = jnp.zeros((L,), jnp.int32)
                for q in range(D // L):
                    if src is None:
                        vq = plsc.load_gather(tailbuf,
                                              [q * L + lanes, col])
                    else:
                        vq = plsc.load_gather(
                            src, [zero16, q * L + lanes, col])
                    rowstage[0, pl.ds(q * L, L)] = vq
                return pos, 0

            lax.fori_loop(0, B // L, slow_chunk, 0)

    do_table(eu_h, tu_h, su_h, u_h)
    do_table(ev_h, tv_h, sv_h, v_h)


def _combine_body(u2_h, v2_h, su_h, sv_h, bu_h, bv_h, out_h,
                  idx_u, idx_v, sbu, sbv, b_u, b_v, out_v, sem):
    wid = lax.axis_index("s") * NC + lax.axis_index("c")
    base = wid * BPW
    lanes = lax.iota(jnp.int32, L)

    pltpu.sync_copy(u2_h.at[pl.ds(wid * 4, 4)], idx_u)
    pltpu.sync_copy(v2_h.at[pl.ds(wid * 4, 4)], idx_v)

    copies = []
    for j in range(4):
        dst = pl.ds(j * 128, 128)
        copies.append(pltpu.async_copy(bu_h.at[idx_u.at[j]],
                                       b_u.at[dst], sem))
        copies.append(pltpu.async_copy(bv_h.at[idx_v.at[j]],
                                       b_v.at[dst], sem))
    for c in copies:
        c.wait()

    for h in range(2):
        hb = base + h * 256
        pltpu.sync_copy(su_h.at[pl.ds(hb, 256)], sbu)
        pltpu.sync_copy(sv_h.at[pl.ds(hb, 256)], sbv)

        def group(g, _):
            rows = g * L + lanes
            boff = h * 256 + g * L
            acc = b_u[pl.ds(boff, L)] + b_v[pl.ds(boff, L)]
            for d in range(D):
                col = jnp.full((L,), d, jnp.int32)
                acc = acc + plsc.load_gather(sbu, [rows, col]) * \
                    plsc.load_gather(sbv, [rows, col])
            out_v[pl.ds(h * 256 + g * L, L)] = jnp.clip(acc, 0.0, 5.0)
            return 0

        lax.fori_loop(0, 256 // L, group, 0)

    pltpu.sync_copy(out_v, out_h.at[pl.ds(base, BPW)])


_extract = functools.partial(
    pl.kernel,
    out_type=(jax.ShapeDtypeStruct((B, D), jnp.float32),
              jax.ShapeDtypeStruct((B, D), jnp.float32)),
    mesh=plsc.VectorSubcoreMesh(core_axis_name="c", subcore_axis_name="s"),
    scratch_types=[
        pltpu.VMEM((B,), jnp.int32),
        pltpu.VMEM((D, 64), jnp.float32),
        pltpu.VMEM((KR, D, 128), jnp.float32),
        pltpu.VMEM((16, 128), jnp.float32),
        pltpu.VMEM((256,), jnp.int32),
        pltpu.SMEM((257,), jnp.int32),
        pltpu.SMEM((256,), jnp.int32),
        pltpu.SMEM((CAP,), jnp.int32),
        pltpu.SMEM((8,), jnp.int32),
        pltpu.SemaphoreType.DMA,
        pltpu.SemaphoreType.DMA,
    ],
    compiler_params=pltpu.CompilerParams(needs_layout_passes=False),
)(_extract_body)


_combine = functools.partial(
    pl.kernel,
    out_type=jax.ShapeDtypeStruct((B,), jnp.float32),
    mesh=plsc.VectorSubcoreMesh(core_axis_name="c", subcore_axis_name="s"),
    scratch_types=[
        pltpu.VMEM((4, 128), jnp.int32),
        pltpu.VMEM((4, 128), jnp.int32),
        pltpu.VMEM((256, D), jnp.float32),
        pltpu.VMEM((256, D), jnp.float32),
        pltpu.VMEM((BPW,), jnp.float32),
        pltpu.VMEM((BPW,), jnp.float32),
        pltpu.VMEM((BPW,), jnp.float32),
        pltpu.SemaphoreType.DMA,
    ],
    compiler_params=pltpu.CompilerParams(needs_layout_passes=False),
)(_combine_body)


def kernel(user, item, embed_user, embed_item, bias_user, bias_item):
    u1 = user.astype(jnp.int32)
    v1 = item.astype(jnp.int32)
    eu_t = embed_user.T
    ev_t = embed_item.T
    tail_u = embed_user[XTAIL:].T
    tail_v = embed_item[XTAIL:].T
    su, sv = _extract(u1, v1, eu_t, ev_t, tail_u, tail_v)
    return _combine(u1.reshape(128, 128), v1.reshape(128, 128), su, sv,
                    bias_user.reshape(-1), bias_item.reshape(-1))
